# baseline (speedup 1.0000x reference)
import functools

import jax
import jax.numpy as jnp
from jax.experimental import pallas as pl
from jax.experimental.pallas import tpu as pltpu

_ALPHA = 16.0
_RANK = 16
_SCALING = _ALPHA / _RANK

_FN = 1024
_FK = 1024
_BM = 1024
_BN = 1024
_BK = 4096


def _fold_kernel(w_ref, b_ref, a_ref, o_ref, *, scaling):
    delta = jax.lax.dot_general(
        b_ref[...].astype(jnp.bfloat16), a_ref[...].astype(jnp.bfloat16),
        (((1,), (0,)), ((), ())), preferred_element_type=jnp.float32)
    o_ref[...] = (w_ref[...] + scaling * delta).astype(jnp.bfloat16)


def _matmul_kernel(x_ref, w_ref, o_ref):
    k = pl.program_id(2)

    @pl.when(k == 0)
    def _store():
        o_ref[...] = jax.lax.dot_general(
            x_ref[...].astype(jnp.bfloat16), w_ref[...],
            (((1,), (1,)), ((), ())), preferred_element_type=jnp.float32)

    @pl.when(k != 0)
    def _accum():
        o_ref[...] += jax.lax.dot_general(
            x_ref[...].astype(jnp.bfloat16), w_ref[...],
            (((1,), (1,)), ((), ())), preferred_element_type=jnp.float32)


@jax.jit
def kernel(x, weight, lora_A, lora_B):
    m, kdim = x.shape
    n = weight.shape[0]
    a0 = lora_A[0, :_RANK, :]
    b0 = lora_B[0, :, :_RANK]


    w_eff = pl.pallas_call(
        functools.partial(_fold_kernel, scaling=_SCALING),
        grid=(n // _FN, kdim // _FK),
        in_specs=[
            pl.BlockSpec((_FN, _FK), lambda j, k: (j, k)),
            pl.BlockSpec((_FN, _RANK), lambda j, k: (j, 0)),
            pl.BlockSpec((_RANK, _FK), lambda j, k: (0, k)),
        ],
        out_specs=pl.BlockSpec((_FN, _FK), lambda j, k: (j, k)),
        out_shape=jax.ShapeDtypeStruct((n, kdim), jnp.bfloat16),
        compiler_params=pltpu.CompilerParams(
            dimension_semantics=("parallel", "parallel"),
        ),
    )(weight, b0, a0)

    return pl.pallas_call(
        _matmul_kernel,
        grid=(m // _BM, n // _BN, kdim // _BK),
        in_specs=[
            pl.BlockSpec((_BM, _BK), lambda i, j, k: (i, k)),
            pl.BlockSpec((_BN, _BK), lambda i, j, k: (j, k)),
        ],
        out_specs=pl.BlockSpec((_BM, _BN), lambda i, j, k: (i, j)),
        out_shape=jax.ShapeDtypeStruct((m, n), jnp.float32),
        compiler_params=pltpu.CompilerParams(
            dimension_semantics=("parallel", "parallel", "arbitrary"),
            vmem_limit_bytes=67108864,
        ),
    )(x, w_eff)


# speedup vs baseline: 1.0129x; 1.0129x over previous
import functools

import jax
import jax.numpy as jnp
from jax.experimental import pallas as pl
from jax.experimental.pallas import tpu as pltpu

_ALPHA = 16.0
_RANK = 16
_SCALING = _ALPHA / _RANK

_FN = 1024
_FK = 1024
_BM = 1024
_BN = 2048
_BK = 2048


def _fold_kernel(w_ref, b_ref, a_ref, o_ref, *, scaling):
    delta = jax.lax.dot_general(
        b_ref[...].astype(jnp.bfloat16), a_ref[...].astype(jnp.bfloat16),
        (((1,), (0,)), ((), ())), preferred_element_type=jnp.float32)
    o_ref[...] = (w_ref[...] + scaling * delta).astype(jnp.bfloat16)


def _matmul_kernel(x_ref, w_ref, o_ref):
    k = pl.program_id(2)

    @pl.when(k == 0)
    def _store():
        o_ref[...] = jax.lax.dot_general(
            x_ref[...].astype(jnp.bfloat16), w_ref[...],
            (((1,), (1,)), ((), ())), preferred_element_type=jnp.float32)

    @pl.when(k != 0)
    def _accum():
        o_ref[...] += jax.lax.dot_general(
            x_ref[...].astype(jnp.bfloat16), w_ref[...],
            (((1,), (1,)), ((), ())), preferred_element_type=jnp.float32)


@jax.jit
def kernel(x, weight, lora_A, lora_B):
    m, kdim = x.shape
    n = weight.shape[0]
    a0 = lora_A[0, :_RANK, :]
    b0 = lora_B[0, :, :_RANK]


    w_eff = pl.pallas_call(
        functools.partial(_fold_kernel, scaling=_SCALING),
        grid=(n // _FN, kdim // _FK),
        in_specs=[
            pl.BlockSpec((_FN, _FK), lambda j, k: (j, k)),
            pl.BlockSpec((_FN, _RANK), lambda j, k: (j, 0)),
            pl.BlockSpec((_RANK, _FK), lambda j, k: (0, k)),
        ],
        out_specs=pl.BlockSpec((_FN, _FK), lambda j, k: (j, k)),
        out_shape=jax.ShapeDtypeStruct((n, kdim), jnp.bfloat16),
        compiler_params=pltpu.CompilerParams(
            dimension_semantics=("parallel", "parallel"),
        ),
    )(weight, b0, a0)

    return pl.pallas_call(
        _matmul_kernel,
        grid=(m // _BM, n // _BN, kdim // _BK),
        in_specs=[
            pl.BlockSpec((_BM, _BK), lambda i, j, k: (i, k)),
            pl.BlockSpec((_BN, _BK), lambda i, j, k: (j, k)),
        ],
        out_specs=pl.BlockSpec((_BM, _BN), lambda i, j, k: (i, j)),
        out_shape=jax.ShapeDtypeStruct((m, n), jnp.float32),
        compiler_params=pltpu.CompilerParams(
            dimension_semantics=("arbitrary", "arbitrary", "arbitrary"),
            vmem_limit_bytes=67108864,
        ),
    )(x, w_eff)


# speedup vs baseline: 1.0130x; 1.0001x over previous
import functools

import jax
import jax.numpy as jnp
from jax.experimental import pallas as pl
from jax.experimental.pallas import tpu as pltpu

_ALPHA = 16.0
_RANK = 16
_SCALING = _ALPHA / _RANK

_FN = 2048
_FK = 2048
_BM = 1024
_BN = 2048
_BK = 2048


def _fold_kernel(w_ref, b_ref, a_ref, o_ref, *, scaling):
    delta = jax.lax.dot_general(
        b_ref[...].astype(jnp.bfloat16), a_ref[...].astype(jnp.bfloat16),
        (((1,), (0,)), ((), ())), preferred_element_type=jnp.float32)
    o_ref[...] = (w_ref[...] + scaling * delta).astype(jnp.bfloat16)


def _matmul_kernel(x_ref, w_ref, o_ref):
    k = pl.program_id(2)

    @pl.when(k == 0)
    def _store():
        o_ref[...] = jax.lax.dot_general(
            x_ref[...].astype(jnp.bfloat16), w_ref[...],
            (((1,), (1,)), ((), ())), preferred_element_type=jnp.float32)

    @pl.when(k != 0)
    def _accum():
        o_ref[...] += jax.lax.dot_general(
            x_ref[...].astype(jnp.bfloat16), w_ref[...],
            (((1,), (1,)), ((), ())), preferred_element_type=jnp.float32)


@jax.jit
def kernel(x, weight, lora_A, lora_B):
    m, kdim = x.shape
    n = weight.shape[0]
    a0 = lora_A[0, :_RANK, :]
    b0 = lora_B[0, :, :_RANK]


    w_eff = pl.pallas_call(
        functools.partial(_fold_kernel, scaling=_SCALING),
        grid=(n // _FN, kdim // _FK),
        in_specs=[
            pl.BlockSpec((_FN, _FK), lambda j, k: (j, k)),
            pl.BlockSpec((_FN, _RANK), lambda j, k: (j, 0)),
            pl.BlockSpec((_RANK, _FK), lambda j, k: (0, k)),
        ],
        out_specs=pl.BlockSpec((_FN, _FK), lambda j, k: (j, k)),
        out_shape=jax.ShapeDtypeStruct((n, kdim), jnp.bfloat16),
        compiler_params=pltpu.CompilerParams(
            dimension_semantics=("parallel", "parallel"),
            vmem_limit_bytes=67108864,
        ),
    )(weight, b0, a0)

    return pl.pallas_call(
        _matmul_kernel,
        grid=(m // _BM, n // _BN, kdim // _BK),
        in_specs=[
            pl.BlockSpec((_BM, _BK), lambda i, j, k: (i, k)),
            pl.BlockSpec((_BN, _BK), lambda i, j, k: (j, k)),
        ],
        out_specs=pl.BlockSpec((_BM, _BN), lambda i, j, k: (i, j)),
        out_shape=jax.ShapeDtypeStruct((m, n), jnp.float32),
        compiler_params=pltpu.CompilerParams(
            dimension_semantics=("arbitrary", "arbitrary", "arbitrary"),
            vmem_limit_bytes=67108864,
        ),
    )(x, w_eff)
